# Initial kernel scaffold; baseline (speedup 1.0000x reference)
import jax
import jax.numpy as jnp
from jax.experimental import pallas as pl


def kernel(features, edge_index, W_pool1, b_pool1, W_self1, b_self1, W_neigh1, b_neigh1, W_pool2, b_pool2, W_self2, b_self2, W_neigh2, b_neigh2):
    raise NotImplementedError("write your pallas kernel here")


# speedup vs baseline: 1.0496x; 1.0496x over previous
import jax
import jax.numpy as jnp
from jax.experimental import pallas as pl
from jax.experimental.pallas import tpu as pltpu

N = 10000
E = 320000
D_IN = 128
D_HID = 16
D_OUT = 40


def _stage1_body(f_ref, wp_ref, bp_ref, ws_ref, bs_ref, hpool_ref, self1_ref):
    f = f_ref[...]
    hpool_ref[...] = jax.nn.relu(
        jnp.dot(f, wp_ref[...], preferred_element_type=jnp.float32) + bp_ref[...]
    )
    self1_ref[...] = jnp.dot(f, ws_ref[...], preferred_element_type=jnp.float32) + bs_ref[...]


def _stage2_body(self1_ref, neigh1_ref, wn1_ref, bn1_ref, wp2_ref, bp2_ref,
                 ws2_ref, bs2_ref, hpool2_ref, self2_ref):
    h = jax.nn.relu(
        self1_ref[...]
        + jnp.dot(neigh1_ref[...], wn1_ref[...], preferred_element_type=jnp.float32)
        + bn1_ref[...]
    )
    hpool2_ref[...] = jax.nn.relu(
        jnp.dot(h, wp2_ref[...], preferred_element_type=jnp.float32) + bp2_ref[...]
    )
    self2_ref[...] = jnp.dot(h, ws2_ref[...], preferred_element_type=jnp.float32) + bs2_ref[...]


def _stage3_body(self2_ref, neigh2_ref, wn2_ref, bn2_ref, out_ref):
    out_ref[...] = (
        self2_ref[...]
        + jnp.dot(neigh2_ref[...], wn2_ref[...], preferred_element_type=jnp.float32)
        + bn2_ref[...]
    )


def kernel(features, edge_index, W_pool1, b_pool1, W_self1, b_self1, W_neigh1, b_neigh1,
           W_pool2, b_pool2, W_self2, b_self2, W_neigh2, b_neigh2):
    src = edge_index[0]
    dst = edge_index[1]

    hpool1, self1 = pl.pallas_call(
        _stage1_body,
        out_shape=(
            jax.ShapeDtypeStruct((N, D_IN), jnp.float32),
            jax.ShapeDtypeStruct((N, D_HID), jnp.float32),
        ),
    )(features, W_pool1, b_pool1, W_self1, b_self1)

    msg = jnp.take(hpool1, src, axis=0)
    neigh1 = jax.ops.segment_max(msg, dst, num_segments=N)
    neigh1 = jnp.maximum(neigh1, 0.0)

    hpool2, self2 = pl.pallas_call(
        _stage2_body,
        out_shape=(
            jax.ShapeDtypeStruct((N, D_HID), jnp.float32),
            jax.ShapeDtypeStruct((N, D_OUT), jnp.float32),
        ),
    )(self1, neigh1, W_neigh1, b_neigh1, W_pool2, b_pool2, W_self2, b_self2)

    msg2 = jnp.take(hpool2, src, axis=0)
    neigh2 = jax.ops.segment_max(msg2, dst, num_segments=N)
    neigh2 = jnp.maximum(neigh2, 0.0)

    out = pl.pallas_call(
        _stage3_body,
        out_shape=jax.ShapeDtypeStruct((N, D_OUT), jnp.float32),
    )(self2, neigh2, W_neigh2, b_neigh2)
    return out


# speedup vs baseline: 3.2410x; 3.0877x over previous
import functools

import jax
import jax.numpy as jnp
from jax import lax
from jax.experimental import pallas as pl
from jax.experimental.pallas import tpu as pltpu
from jax.experimental.pallas import tpu_sc as plsc

N = 10000
E = 320000
D_IN = 128
D_HID = 16
D_OUT = 40

NC = 2
NS = 16
NW = NC * NS
BINW = 320
NPAD = NW * BINW
MAGIC = 3277
MSHIFT = 20
SLOT = 16384
CHUNK = 20000
KROW = 128


def _worker_id():
    return lax.axis_index("s") * NC + lax.axis_index("c")


def _bin_body(src_hbm, dst_hbm, qsrc_hbm, qdst_hbm, qcnt_hbm,
              sbuf, dbuf, qsrc, qdst, cntv):
    w = _worker_id()

    @pl.loop(0, SLOT // 16)
    def _zinit(i):
        z = jnp.zeros((16,), jnp.int32)
        qsrc[pl.ds(i * 16, 16)] = z
        qdst[pl.ds(i * 16, 16)] = z

    qn0 = jnp.zeros((16,), jnp.int32)

    def scan_chunk(g, qn):
        pltpu.sync_copy(src_hbm.at[pl.ds(g * CHUNK, CHUNK)], sbuf)
        pltpu.sync_copy(dst_hbm.at[pl.ds(g * CHUNK, CHUNK)], dbuf)

        @pl.loop(0, CHUNK // 16, init_carry=qn)
        def inner(i, qn):
            d = dbuf[pl.ds(i * 16, 16)]
            s = sbuf[pl.ds(i * 16, 16)]
            b = (d * MAGIC) >> MSHIFT
            m = b == w
            mi = m.astype(jnp.int32)
            rank = plsc.cumsum(mi) - mi
            pos = qn + rank
            m2 = m & (pos < SLOT)
            plsc.store_scatter(qsrc, [pos], s, mask=m2)
            plsc.store_scatter(qdst, [pos], d, mask=m2)
            return qn + plsc.all_reduce_population_count(m)

        return inner

    qn = qn0
    for g in range(E // CHUNK):
        qn = scan_chunk(g, qn)

    cntv[pl.ds(0, 16)] = jnp.minimum(qn, SLOT)
    pltpu.sync_copy(qsrc, qsrc_hbm.at[pl.ds(w * SLOT, SLOT)])
    pltpu.sync_copy(qdst, qdst_hbm.at[pl.ds(w * SLOT, SLOT)])
    pltpu.sync_copy(cntv, qcnt_hbm.at[pl.ds(w * 16, 16)])


_SC_PARAMS = pltpu.CompilerParams(needs_layout_passes=False, use_tc_tiling_on_sc=False)


def _bin_edges(src, dst):
    mesh = plsc.VectorSubcoreMesh(core_axis_name="c", subcore_axis_name="s")
    return pl.kernel(
        _bin_body,
        compiler_params=_SC_PARAMS,
        out_type=(
            jax.ShapeDtypeStruct((NW * SLOT,), jnp.int32),
            jax.ShapeDtypeStruct((NW * SLOT,), jnp.int32),
            jax.ShapeDtypeStruct((NW * 16,), jnp.int32),
        ),
        mesh=mesh,
        scratch_types=[
            pltpu.VMEM((CHUNK,), jnp.int32),
            pltpu.VMEM((CHUNK,), jnp.int32),
            pltpu.VMEM((SLOT,), jnp.int32),
            pltpu.VMEM((SLOT,), jnp.int32),
            pltpu.VMEM((16,), jnp.int32),
        ],
    )(src, dst)


def _seg_body(table_hbm, qsrc_hbm, qdst_hbm, qcnt_hbm, out_hbm,
              qs, qd, rows, acc, cntv, sem0, sem1, *, D):
    DV = D // 16
    w = _worker_id()
    sems = (sem0, sem1)

    @pl.loop(0, BINW + 1)
    def _zinit(r):
        for j in range(DV):
            acc[r, pl.ds(j * 16, 16)] = jnp.zeros((16,), jnp.float32)

    pltpu.sync_copy(qcnt_hbm.at[pl.ds(w * 16, 16)], cntv)
    cnt = cntv[pl.ds(0, 16)][0]
    nch = (cnt + KROW - 1) >> 7

    def fire(cc, b):
        pltpu.sync_copy(qsrc_hbm.at[pl.ds(w * SLOT + cc * KROW, KROW)], qs.at[b])
        pltpu.sync_copy(qdst_hbm.at[pl.ds(w * SLOT + cc * KROW, KROW)], qd.at[b])
        pltpu.async_copy(table_hbm.at[qs.at[b]], rows.at[b], sems[b])

    def wait(b):
        pltpu.make_async_copy(table_hbm.at[qs.at[b]], rows.at[b], sems[b]).wait()

    @pl.when(nch > 0)
    def _():
        fire(0, 0)

    nch2 = ((nch + 1) >> 1) << 1

    @pl.loop(0, nch2, step=2)
    def _outer(c2):
        for b in (0, 1):
            cc = c2 + b

            @pl.when(cc < nch)
            def _():
                @pl.when(cc + 1 < nch)
                def _():
                    fire(cc + 1, 1 - b)

                wait(b)

                @pl.loop(0, KROW // 16)
                def _vec(v):
                    ldv = qd[b, pl.ds(v * 16, 16)] - BINW * w
                    ev = cc * KROW + v * 16 + lax.iota(jnp.int32, 16)
                    ldv = jnp.where(ev < cnt, ldv, BINW)
                    for l in range(16):
                        ld = ldv[l]
                        e = v * 16 + l
                        for j in range(DV):
                            sl = pl.ds(j * 16, 16)
                            acc[ld, sl] = jnp.maximum(acc[ld, sl], rows[b, e, sl])

    pltpu.sync_copy(acc.at[pl.ds(0, BINW)], out_hbm.at[pl.ds(BINW * w, BINW)])


def _segment_max(table, qsrc, qdst, qcnt, D):
    mesh = plsc.VectorSubcoreMesh(core_axis_name="c", subcore_axis_name="s")
    body = functools.partial(_seg_body, D=D)
    return pl.kernel(
        body,
        compiler_params=_SC_PARAMS,
        out_type=jax.ShapeDtypeStruct((NPAD, D), jnp.float32),
        mesh=mesh,
        scratch_types=[
            pltpu.VMEM((2, KROW), jnp.int32),
            pltpu.VMEM((2, KROW), jnp.int32),
            pltpu.VMEM((2, KROW, D), jnp.float32),
            pltpu.VMEM((BINW + 1, D), jnp.float32),
            pltpu.VMEM((16,), jnp.int32),
            pltpu.SemaphoreType.DMA,
            pltpu.SemaphoreType.DMA,
        ],
    )(table, qsrc, qdst, qcnt)


def _stage1_body(f_ref, wp_ref, bp_ref, ws_ref, bs_ref, hpool_ref, self1_ref):
    f = f_ref[...]
    hpool_ref[...] = jax.nn.relu(
        jnp.dot(f, wp_ref[...], preferred_element_type=jnp.float32) + bp_ref[...]
    )
    self1_ref[...] = jnp.dot(f, ws_ref[...], preferred_element_type=jnp.float32) + bs_ref[...]


def _stage2_body(self1_ref, neigh1_ref, wn1_ref, bn1_ref, wp2_ref, bp2_ref,
                 ws2_ref, bs2_ref, hpool2_ref, self2_ref):
    h = jax.nn.relu(
        self1_ref[...]
        + jnp.dot(neigh1_ref[...], wn1_ref[...], preferred_element_type=jnp.float32)
        + bn1_ref[...]
    )
    hpool2_ref[...] = jax.nn.relu(
        jnp.dot(h, wp2_ref[...], preferred_element_type=jnp.float32) + bp2_ref[...]
    )
    self2_ref[...] = jnp.dot(h, ws2_ref[...], preferred_element_type=jnp.float32) + bs2_ref[...]


def _stage3_body(self2_ref, neigh2_ref, wn2_ref, bn2_ref, out_ref):
    out_ref[...] = (
        self2_ref[...]
        + jnp.dot(neigh2_ref[...], wn2_ref[...], preferred_element_type=jnp.float32)
        + bn2_ref[...]
    )


def kernel(features, edge_index, W_pool1, b_pool1, W_self1, b_self1, W_neigh1, b_neigh1,
           W_pool2, b_pool2, W_self2, b_self2, W_neigh2, b_neigh2):
    src = edge_index[0]
    dst = edge_index[1]

    qsrc, qdst, qcnt = _bin_edges(src, dst)

    hpool1, self1 = pl.pallas_call(
        _stage1_body,
        out_shape=(
            jax.ShapeDtypeStruct((N, D_IN), jnp.float32),
            jax.ShapeDtypeStruct((N, D_HID), jnp.float32),
        ),
    )(features, W_pool1, b_pool1, W_self1, b_self1)

    neigh1 = _segment_max(hpool1, qsrc, qdst, qcnt, D_IN)[:N]

    hpool2, self2 = pl.pallas_call(
        _stage2_body,
        out_shape=(
            jax.ShapeDtypeStruct((N, D_HID), jnp.float32),
            jax.ShapeDtypeStruct((N, D_OUT), jnp.float32),
        ),
    )(self1, neigh1, W_neigh1, b_neigh1, W_pool2, b_pool2, W_self2, b_self2)

    neigh2 = _segment_max(hpool2, qsrc, qdst, qcnt, D_HID)[:N]

    out = pl.pallas_call(
        _stage3_body,
        out_shape=jax.ShapeDtypeStruct((N, D_OUT), jnp.float32),
    )(self2, neigh2, W_neigh2, b_neigh2)
    return out


# speedup vs baseline: 3.9817x; 1.2285x over previous
import functools

import jax
import jax.numpy as jnp
from jax import lax
from jax.experimental import pallas as pl
from jax.experimental.pallas import tpu as pltpu
from jax.experimental.pallas import tpu_sc as plsc

N = 10000
E = 320000
D_IN = 128
D_HID = 16
D_OUT = 40

NC = 2
NS = 16
NW = NC * NS
BINW = 320
NPAD = NW * BINW
MAGIC = 3277
MSHIFT = 20
SLOT = 16384
CHUNK = 20000
KROW = 128


def _worker_id():
    return lax.axis_index("s") * NC + lax.axis_index("c")


def _bin_body(src_hbm, dst_hbm, qsrc_hbm, qdst_hbm, qcnt_hbm,
              sbuf, dbuf, qsrc, qdst, cntv):
    w = _worker_id()

    @pl.loop(0, SLOT // 16)
    def _zinit(i):
        z = jnp.zeros((16,), jnp.int32)
        qsrc[pl.ds(i * 16, 16)] = z
        qdst[pl.ds(i * 16, 16)] = z

    qn0 = jnp.zeros((16,), jnp.int32)

    def scan_chunk(g, qn):
        pltpu.sync_copy(src_hbm.at[pl.ds(g * CHUNK, CHUNK)], sbuf)
        pltpu.sync_copy(dst_hbm.at[pl.ds(g * CHUNK, CHUNK)], dbuf)

        @pl.loop(0, CHUNK // 16, init_carry=qn)
        def inner(i, qn):
            d = dbuf[pl.ds(i * 16, 16)]
            s = sbuf[pl.ds(i * 16, 16)]
            b = (d * MAGIC) >> MSHIFT
            m = b == w
            mi = m.astype(jnp.int32)
            rank = plsc.cumsum(mi) - mi
            pos = qn + rank
            m2 = m & (pos < SLOT)
            plsc.store_scatter(qsrc, [pos], s, mask=m2)
            plsc.store_scatter(qdst, [pos], d, mask=m2)
            return qn + plsc.all_reduce_population_count(m)

        return inner

    qn = qn0
    for g in range(E // CHUNK):
        qn = scan_chunk(g, qn)

    cntv[pl.ds(0, 16)] = jnp.minimum(qn, SLOT)
    pltpu.sync_copy(qsrc, qsrc_hbm.at[pl.ds(w * SLOT, SLOT)])
    pltpu.sync_copy(qdst, qdst_hbm.at[pl.ds(w * SLOT, SLOT)])
    pltpu.sync_copy(cntv, qcnt_hbm.at[pl.ds(w * 16, 16)])


_SC_PARAMS = pltpu.CompilerParams(needs_layout_passes=False, use_tc_tiling_on_sc=False)


def _bin_edges(src, dst):
    mesh = plsc.VectorSubcoreMesh(core_axis_name="c", subcore_axis_name="s")
    return pl.kernel(
        _bin_body,
        compiler_params=_SC_PARAMS,
        out_type=(
            jax.ShapeDtypeStruct((NW * SLOT,), jnp.int32),
            jax.ShapeDtypeStruct((NW * SLOT,), jnp.int32),
            jax.ShapeDtypeStruct((NW * 16,), jnp.int32),
        ),
        mesh=mesh,
        scratch_types=[
            pltpu.VMEM((CHUNK,), jnp.int32),
            pltpu.VMEM((CHUNK,), jnp.int32),
            pltpu.VMEM((SLOT,), jnp.int32),
            pltpu.VMEM((SLOT,), jnp.int32),
            pltpu.VMEM((16,), jnp.int32),
        ],
    )(src, dst)


def _seg_body(table_hbm, qsrc_hbm, qdst_hbm, qcnt_hbm, out_hbm,
              qs, qd, rows, acc, cntv, sem0, sem1, *, D):
    DV = D // 16
    w = _worker_id()
    sems = (sem0, sem1)

    @pl.loop(0, BINW + 1)
    def _zinit(r):
        for j in range(DV):
            acc[r, pl.ds(j * 16, 16)] = jnp.zeros((16,), jnp.float32)

    pltpu.sync_copy(qcnt_hbm.at[pl.ds(w * 16, 16)], cntv)
    cnt = cntv[pl.ds(0, 16)][0]
    nch = (cnt + KROW - 1) >> 7

    def fire(cc, b):
        pltpu.sync_copy(qsrc_hbm.at[pl.ds(w * SLOT + cc * KROW, KROW)], qs.at[b])
        pltpu.sync_copy(qdst_hbm.at[pl.ds(w * SLOT + cc * KROW, KROW)], qd.at[b])
        pltpu.async_copy(table_hbm.at[qs.at[b]], rows.at[b], sems[b])

    def wait(b):
        pltpu.make_async_copy(table_hbm.at[qs.at[b]], rows.at[b], sems[b]).wait()

    @pl.when(nch > 0)
    def _():
        fire(0, 0)

    nch2 = ((nch + 1) >> 1) << 1

    @pl.loop(0, nch2, step=2)
    def _outer(c2):
        for b in (0, 1):
            cc = c2 + b

            @pl.when(cc < nch)
            def _():
                @pl.when(cc + 1 < nch)
                def _():
                    fire(cc + 1, 1 - b)

                wait(b)

                @pl.loop(0, KROW // 16)
                def _vec(v):
                    ldv = qd[b, pl.ds(v * 16, 16)] - BINW * w
                    ev = cc * KROW + v * 16 + lax.iota(jnp.int32, 16)
                    ldv = jnp.where(ev < cnt, ldv, BINW)
                    for l in range(16):
                        ld = ldv[l]
                        e = v * 16 + l
                        cur = [acc[ld, pl.ds(j * 16, 16)] for j in range(DV)]
                        rv = [rows[b, e, pl.ds(j * 16, 16)] for j in range(DV)]
                        for j in range(DV):
                            acc[ld, pl.ds(j * 16, 16)] = jnp.maximum(cur[j], rv[j])

    pltpu.sync_copy(acc.at[pl.ds(0, BINW)], out_hbm.at[pl.ds(BINW * w, BINW)])


def _segment_max(table, qsrc, qdst, qcnt, D):
    mesh = plsc.VectorSubcoreMesh(core_axis_name="c", subcore_axis_name="s")
    body = functools.partial(_seg_body, D=D)
    return pl.kernel(
        body,
        compiler_params=_SC_PARAMS,
        out_type=jax.ShapeDtypeStruct((NPAD, D), jnp.float32),
        mesh=mesh,
        scratch_types=[
            pltpu.VMEM((2, KROW), jnp.int32),
            pltpu.VMEM((2, KROW), jnp.int32),
            pltpu.VMEM((2, KROW, D), jnp.float32),
            pltpu.VMEM((BINW + 1, D), jnp.float32),
            pltpu.VMEM((16,), jnp.int32),
            pltpu.SemaphoreType.DMA,
            pltpu.SemaphoreType.DMA,
        ],
    )(table, qsrc, qdst, qcnt)


def _stage1_body(f_ref, wp_ref, bp_ref, ws_ref, bs_ref, hpool_ref, self1_ref):
    f = f_ref[...]
    hpool_ref[...] = jax.nn.relu(
        jnp.dot(f, wp_ref[...], preferred_element_type=jnp.float32) + bp_ref[...]
    )
    self1_ref[...] = jnp.dot(f, ws_ref[...], preferred_element_type=jnp.float32) + bs_ref[...]


def _stage2_body(self1_ref, neigh1_ref, wn1_ref, bn1_ref, wp2_ref, bp2_ref,
                 ws2_ref, bs2_ref, hpool2_ref, self2_ref):
    h = jax.nn.relu(
        self1_ref[...]
        + jnp.dot(neigh1_ref[...], wn1_ref[...], preferred_element_type=jnp.float32)
        + bn1_ref[...]
    )
    hpool2_ref[...] = jax.nn.relu(
        jnp.dot(h, wp2_ref[...], preferred_element_type=jnp.float32) + bp2_ref[...]
    )
    self2_ref[...] = jnp.dot(h, ws2_ref[...], preferred_element_type=jnp.float32) + bs2_ref[...]


def _stage3_body(self2_ref, neigh2_ref, wn2_ref, bn2_ref, out_ref):
    out_ref[...] = (
        self2_ref[...]
        + jnp.dot(neigh2_ref[...], wn2_ref[...], preferred_element_type=jnp.float32)
        + bn2_ref[...]
    )


def kernel(features, edge_index, W_pool1, b_pool1, W_self1, b_self1, W_neigh1, b_neigh1,
           W_pool2, b_pool2, W_self2, b_self2, W_neigh2, b_neigh2):
    src = edge_index[0]
    dst = edge_index[1]

    qsrc, qdst, qcnt = _bin_edges(src, dst)

    hpool1, self1 = pl.pallas_call(
        _stage1_body,
        out_shape=(
            jax.ShapeDtypeStruct((N, D_IN), jnp.float32),
            jax.ShapeDtypeStruct((N, D_HID), jnp.float32),
        ),
    )(features, W_pool1, b_pool1, W_self1, b_self1)

    neigh1 = _segment_max(hpool1, qsrc, qdst, qcnt, D_IN)[:N]

    hpool2, self2 = pl.pallas_call(
        _stage2_body,
        out_shape=(
            jax.ShapeDtypeStruct((N, D_HID), jnp.float32),
            jax.ShapeDtypeStruct((N, D_OUT), jnp.float32),
        ),
    )(self1, neigh1, W_neigh1, b_neigh1, W_pool2, b_pool2, W_self2, b_self2)

    neigh2 = _segment_max(hpool2, qsrc, qdst, qcnt, D_HID)[:N]

    out = pl.pallas_call(
        _stage3_body,
        out_shape=jax.ShapeDtypeStruct((N, D_OUT), jnp.float32),
    )(self2, neigh2, W_neigh2, b_neigh2)
    return out


# speedup vs baseline: 4.4319x; 1.1131x over previous
import functools

import jax
import jax.numpy as jnp
from jax import lax
from jax.experimental import pallas as pl
from jax.experimental.pallas import tpu as pltpu
from jax.experimental.pallas import tpu_sc as plsc

N = 10000
E = 320000
D_IN = 128
D_HID = 16
D_OUT = 40

NC = 2
NS = 16
NW = NC * NS
BINW = 320
NPAD = NW * BINW
MAGIC = 3277
MSHIFT = 20
SLOT = 16384
CHUNK = 20000
KROW = 128


def _worker_id():
    return lax.axis_index("s") * NC + lax.axis_index("c")


LCAP = 1024


def _bin_body(src_hbm, dst_hbm, qsrc_hbm, qdst_hbm, qcnt_hbm,
              sbuf, dbuf, qsl, qdl, qsrc, qdst, cntv):
    w = _worker_id()
    jd = 320 * w + 320
    zv = jnp.zeros((16,), jnp.int32)
    jdv = zv + jd

    @pl.loop(0, LCAP)
    def _jinit(i):
        qsl[pl.ds(i * 16, 16)] = zv
        qdl[pl.ds(i * 16, 16)] = jdv
        qsrc[pl.ds(i * 16, 16)] = zv
        qdst[pl.ds(i * 16, 16)] = jdv

    lb = lax.iota(jnp.int32, 16) * LCAP

    def scan_chunk(g, qn):
        pltpu.sync_copy(src_hbm.at[pl.ds(g * CHUNK, CHUNK)], sbuf)
        pltpu.sync_copy(dst_hbm.at[pl.ds(g * CHUNK, CHUNK)], dbuf)

        @pl.loop(0, CHUNK // 16, init_carry=qn, unroll=2)
        def inner(i, qn):
            d = dbuf[pl.ds(i * 16, 16)]
            s = sbuf[pl.ds(i * 16, 16)]
            b = (d * MAGIC) >> MSHIFT
            m = (b == w) & (qn < LCAP)
            pos = lb + qn
            plsc.store_scatter(qsl, [pos], s, mask=m)
            plsc.store_scatter(qdl, [pos], d, mask=m)
            return qn + m.astype(jnp.int32)

        return inner

    qn = jnp.zeros((16,), jnp.int32)
    for g in range(E // CHUNK):
        qn = scan_chunk(g, qn)

    off = 0
    for l in range(16):
        cl = qn[l]
        nv = (cl + 15) >> 4

        off_now = off

        @pl.loop(0, nv)
        def _copy(v):
            qsrc[pl.ds(off_now + v * 16, 16)] = qsl[pl.ds(l * LCAP + v * 16, 16)]
            qdst[pl.ds(off_now + v * 16, 16)] = qdl[pl.ds(l * LCAP + v * 16, 16)]

        off = off_now + nv * 16

    cntv[pl.ds(0, 16)] = zv + off
    pltpu.sync_copy(qsrc, qsrc_hbm.at[pl.ds(w * SLOT, SLOT)])
    pltpu.sync_copy(qdst, qdst_hbm.at[pl.ds(w * SLOT, SLOT)])
    pltpu.sync_copy(cntv, qcnt_hbm.at[pl.ds(w * 16, 16)])


_SC_PARAMS = pltpu.CompilerParams(needs_layout_passes=False, use_tc_tiling_on_sc=False)


def _bin_edges(src, dst):
    mesh = plsc.VectorSubcoreMesh(core_axis_name="c", subcore_axis_name="s")
    return pl.kernel(
        _bin_body,
        compiler_params=_SC_PARAMS,
        out_type=(
            jax.ShapeDtypeStruct((NW * SLOT,), jnp.int32),
            jax.ShapeDtypeStruct((NW * SLOT,), jnp.int32),
            jax.ShapeDtypeStruct((NW * 16,), jnp.int32),
        ),
        mesh=mesh,
        scratch_types=[
            pltpu.VMEM((CHUNK,), jnp.int32),
            pltpu.VMEM((CHUNK,), jnp.int32),
            pltpu.VMEM((16 * LCAP,), jnp.int32),
            pltpu.VMEM((16 * LCAP,), jnp.int32),
            pltpu.VMEM((SLOT,), jnp.int32),
            pltpu.VMEM((SLOT,), jnp.int32),
            pltpu.VMEM((16,), jnp.int32),
        ],
    )(src, dst)


def _seg_body(table_hbm, qsrc_hbm, qdst_hbm, qcnt_hbm, out_hbm,
              qs, qd, rows, acc, cntv, sem0, sem1, *, D):
    DV = D // 16
    w = _worker_id()
    sems = (sem0, sem1)

    @pl.loop(0, BINW + 1)
    def _zinit(r):
        for j in range(DV):
            acc[r, pl.ds(j * 16, 16)] = jnp.zeros((16,), jnp.float32)

    pltpu.sync_copy(qcnt_hbm.at[pl.ds(w * 16, 16)], cntv)
    cnt = cntv[pl.ds(0, 16)][0]
    nch = (cnt + KROW - 1) >> 7

    def fire(cc, b):
        pltpu.sync_copy(qsrc_hbm.at[pl.ds(w * SLOT + cc * KROW, KROW)], qs.at[b])
        pltpu.sync_copy(qdst_hbm.at[pl.ds(w * SLOT + cc * KROW, KROW)], qd.at[b])
        pltpu.async_copy(table_hbm.at[qs.at[b]], rows.at[b], sems[b])

    def wait(b):
        pltpu.make_async_copy(table_hbm.at[qs.at[b]], rows.at[b], sems[b]).wait()

    @pl.when(nch > 0)
    def _():
        fire(0, 0)

    nch2 = ((nch + 1) >> 1) << 1

    @pl.loop(0, nch2, step=2)
    def _outer(c2):
        for b in (0, 1):
            cc = c2 + b

            @pl.when(cc < nch)
            def _():
                @pl.when(cc + 1 < nch)
                def _():
                    fire(cc + 1, 1 - b)

                wait(b)

                @pl.loop(0, KROW // 16)
                def _vec(v):
                    ldv = qd[b, pl.ds(v * 16, 16)] - BINW * w
                    ev = cc * KROW + v * 16 + lax.iota(jnp.int32, 16)
                    ldv = jnp.where(ev < cnt, ldv, BINW)
                    for l in range(16):
                        ld = ldv[l]
                        e = v * 16 + l
                        cur = [acc[ld, pl.ds(j * 16, 16)] for j in range(DV)]
                        rv = [rows[b, e, pl.ds(j * 16, 16)] for j in range(DV)]
                        for j in range(DV):
                            acc[ld, pl.ds(j * 16, 16)] = jnp.maximum(cur[j], rv[j])

    pltpu.sync_copy(acc.at[pl.ds(0, BINW)], out_hbm.at[pl.ds(BINW * w, BINW)])


def _segment_max(table, qsrc, qdst, qcnt, D):
    mesh = plsc.VectorSubcoreMesh(core_axis_name="c", subcore_axis_name="s")
    body = functools.partial(_seg_body, D=D)
    return pl.kernel(
        body,
        compiler_params=_SC_PARAMS,
        out_type=jax.ShapeDtypeStruct((NPAD, D), jnp.float32),
        mesh=mesh,
        scratch_types=[
            pltpu.VMEM((2, KROW), jnp.int32),
            pltpu.VMEM((2, KROW), jnp.int32),
            pltpu.VMEM((2, KROW, D), jnp.float32),
            pltpu.VMEM((BINW + 1, D), jnp.float32),
            pltpu.VMEM((16,), jnp.int32),
            pltpu.SemaphoreType.DMA,
            pltpu.SemaphoreType.DMA,
        ],
    )(table, qsrc, qdst, qcnt)


def _stage1_body(f_ref, wp_ref, bp_ref, ws_ref, bs_ref, hpool_ref, self1_ref):
    f = f_ref[...]
    hpool_ref[...] = jax.nn.relu(
        jnp.dot(f, wp_ref[...], preferred_element_type=jnp.float32) + bp_ref[...]
    )
    self1_ref[...] = jnp.dot(f, ws_ref[...], preferred_element_type=jnp.float32) + bs_ref[...]


def _stage2_body(self1_ref, neigh1_ref, wn1_ref, bn1_ref, wp2_ref, bp2_ref,
                 ws2_ref, bs2_ref, hpool2_ref, self2_ref):
    h = jax.nn.relu(
        self1_ref[...]
        + jnp.dot(neigh1_ref[...], wn1_ref[...], preferred_element_type=jnp.float32)
        + bn1_ref[...]
    )
    hpool2_ref[...] = jax.nn.relu(
        jnp.dot(h, wp2_ref[...], preferred_element_type=jnp.float32) + bp2_ref[...]
    )
    self2_ref[...] = jnp.dot(h, ws2_ref[...], preferred_element_type=jnp.float32) + bs2_ref[...]


def _stage3_body(self2_ref, neigh2_ref, wn2_ref, bn2_ref, out_ref):
    out_ref[...] = (
        self2_ref[...]
        + jnp.dot(neigh2_ref[...], wn2_ref[...], preferred_element_type=jnp.float32)
        + bn2_ref[...]
    )


def kernel(features, edge_index, W_pool1, b_pool1, W_self1, b_self1, W_neigh1, b_neigh1,
           W_pool2, b_pool2, W_self2, b_self2, W_neigh2, b_neigh2):
    src = edge_index[0]
    dst = edge_index[1]

    qsrc, qdst, qcnt = _bin_edges(src, dst)

    hpool1, self1 = pl.pallas_call(
        _stage1_body,
        out_shape=(
            jax.ShapeDtypeStruct((N, D_IN), jnp.float32),
            jax.ShapeDtypeStruct((N, D_HID), jnp.float32),
        ),
    )(features, W_pool1, b_pool1, W_self1, b_self1)

    neigh1 = _segment_max(hpool1, qsrc, qdst, qcnt, D_IN)[:N]

    hpool2, self2 = pl.pallas_call(
        _stage2_body,
        out_shape=(
            jax.ShapeDtypeStruct((N, D_HID), jnp.float32),
            jax.ShapeDtypeStruct((N, D_OUT), jnp.float32),
        ),
    )(self1, neigh1, W_neigh1, b_neigh1, W_pool2, b_pool2, W_self2, b_self2)

    neigh2 = _segment_max(hpool2, qsrc, qdst, qcnt, D_HID)[:N]

    out = pl.pallas_call(
        _stage3_body,
        out_shape=jax.ShapeDtypeStruct((N, D_OUT), jnp.float32),
    )(self2, neigh2, W_neigh2, b_neigh2)
    return out


# speedup vs baseline: 4.6762x; 1.0551x over previous
import functools

import jax
import jax.numpy as jnp
from jax import lax
from jax.experimental import pallas as pl
from jax.experimental.pallas import tpu as pltpu
from jax.experimental.pallas import tpu_sc as plsc

N = 10000
E = 320000
D_IN = 128
D_HID = 16
D_OUT = 40

NC = 2
NS = 16
NW = NC * NS
BINW = 320
NPAD = NW * BINW
MAGIC = 3277
MSHIFT = 20
SLOT = 16384
CHUNK = 20000
KROW = 128


def _worker_id():
    return lax.axis_index("s") * NC + lax.axis_index("c")


LCAP = 1024


def _bin_body(src_hbm, dst_hbm, qsrc_hbm, qdst_hbm, qcnt_hbm,
              sbuf, dbuf, qsl, qdl, qsrc, qdst, cntv):
    w = _worker_id()
    jd = 320 * w + 320
    zv = jnp.zeros((16,), jnp.int32)
    jdv = zv + jd

    @pl.loop(0, LCAP)
    def _jinit(i):
        qsl[pl.ds(i * 16, 16)] = zv
        qdl[pl.ds(i * 16, 16)] = jdv
        qsrc[pl.ds(i * 16, 16)] = zv
        qdst[pl.ds(i * 16, 16)] = jdv

    lb = lax.iota(jnp.int32, 16) * LCAP

    def scan_chunk(g, qn):
        pltpu.sync_copy(src_hbm.at[pl.ds(g * CHUNK, CHUNK)], sbuf)
        pltpu.sync_copy(dst_hbm.at[pl.ds(g * CHUNK, CHUNK)], dbuf)

        @pl.loop(0, CHUNK // 16, init_carry=qn, unroll=2)
        def inner(i, qn):
            d = dbuf[pl.ds(i * 16, 16)]
            s = sbuf[pl.ds(i * 16, 16)]
            b = (d * MAGIC) >> MSHIFT
            m = (b == w) & (qn < LCAP)
            pos = lb + qn
            plsc.store_scatter(qsl, [pos], s, mask=m)
            plsc.store_scatter(qdl, [pos], d, mask=m)
            return qn + m.astype(jnp.int32)

        return inner

    qn = jnp.zeros((16,), jnp.int32)
    for g in range(E // CHUNK):
        qn = scan_chunk(g, qn)

    off = 0
    for l in range(16):
        cl = qn[l]
        nv = (cl + 15) >> 4

        off_now = off

        @pl.loop(0, nv)
        def _copy(v):
            qsrc[pl.ds(off_now + v * 16, 16)] = qsl[pl.ds(l * LCAP + v * 16, 16)]
            qdst[pl.ds(off_now + v * 16, 16)] = qdl[pl.ds(l * LCAP + v * 16, 16)]

        off = off_now + nv * 16

    cntv[pl.ds(0, 16)] = zv + off
    pltpu.sync_copy(qsrc, qsrc_hbm.at[pl.ds(w * SLOT, SLOT)])
    pltpu.sync_copy(qdst, qdst_hbm.at[pl.ds(w * SLOT, SLOT)])
    pltpu.sync_copy(cntv, qcnt_hbm.at[pl.ds(w * 16, 16)])


_SC_PARAMS = pltpu.CompilerParams(needs_layout_passes=False, use_tc_tiling_on_sc=False)


def _bin_edges(src, dst):
    mesh = plsc.VectorSubcoreMesh(core_axis_name="c", subcore_axis_name="s")
    return pl.kernel(
        _bin_body,
        compiler_params=_SC_PARAMS,
        out_type=(
            jax.ShapeDtypeStruct((NW * SLOT,), jnp.int32),
            jax.ShapeDtypeStruct((NW * SLOT,), jnp.int32),
            jax.ShapeDtypeStruct((NW * 16,), jnp.int32),
        ),
        mesh=mesh,
        scratch_types=[
            pltpu.VMEM((CHUNK,), jnp.int32),
            pltpu.VMEM((CHUNK,), jnp.int32),
            pltpu.VMEM((16 * LCAP,), jnp.int32),
            pltpu.VMEM((16 * LCAP,), jnp.int32),
            pltpu.VMEM((SLOT,), jnp.int32),
            pltpu.VMEM((SLOT,), jnp.int32),
            pltpu.VMEM((16,), jnp.int32),
        ],
    )(src, dst)


def _seg_body(table_hbm, qsrc_hbm, qdst_hbm, qcnt_hbm, out_hbm,
              qs, qd, rows, acc, cntv, rsem0, rsem1, isem0, isem1, *, D):
    DV = D // 16
    w = _worker_id()
    rsems = (rsem0, rsem1)
    isems = (isem0, isem1)

    @pl.loop(0, BINW + 1)
    def _zinit(r):
        for j in range(DV):
            acc[r, pl.ds(j * 16, 16)] = jnp.zeros((16,), jnp.float32)

    pltpu.sync_copy(qcnt_hbm.at[pl.ds(w * 16, 16)], cntv)
    cnt = cntv[pl.ds(0, 16)][0]
    nch = (cnt + KROW - 1) >> 7

    def idx_load(cc, b):
        pltpu.async_copy(qsrc_hbm.at[pl.ds(w * SLOT + cc * KROW, KROW)], qs.at[b], isems[b])
        pltpu.async_copy(qdst_hbm.at[pl.ds(w * SLOT + cc * KROW, KROW)], qd.at[b], isems[b])

    def idx_wait(b):
        pltpu.make_async_copy(qsrc_hbm.at[pl.ds(0, KROW)], qs.at[b], isems[b]).wait()
        pltpu.make_async_copy(qdst_hbm.at[pl.ds(0, KROW)], qd.at[b], isems[b]).wait()

    def gather_fire(b):
        pltpu.async_copy(table_hbm.at[qs.at[b]], rows.at[b], rsems[b])

    def gather_wait(b):
        pltpu.make_async_copy(table_hbm.at[qs.at[b]], rows.at[b], rsems[b]).wait()

    @pl.when(nch > 0)
    def _():
        idx_load(0, 0)

    @pl.when(nch > 1)
    def _():
        idx_load(1, 1)

    @pl.when(nch > 0)
    def _():
        idx_wait(0)
        gather_fire(0)

    nch2 = ((nch + 1) >> 1) << 1

    @pl.loop(0, nch2, step=2)
    def _outer(c2):
        for b in (0, 1):
            cc = c2 + b

            @pl.when(cc < nch)
            def _():
                gather_wait(b)

                @pl.when(cc + 1 < nch)
                def _():
                    idx_wait(1 - b)
                    gather_fire(1 - b)

                @pl.loop(0, KROW // 16)
                def _vec(v):
                    ldv = qd[b, pl.ds(v * 16, 16)] - BINW * w
                    ev = cc * KROW + v * 16 + lax.iota(jnp.int32, 16)
                    ldv = jnp.where(ev < cnt, ldv, BINW)
                    for l in range(16):
                        ld = ldv[l]
                        e = v * 16 + l
                        cur = [acc[ld, pl.ds(j * 16, 16)] for j in range(DV)]
                        rv = [rows[b, e, pl.ds(j * 16, 16)] for j in range(DV)]
                        for j in range(DV):
                            acc[ld, pl.ds(j * 16, 16)] = jnp.maximum(cur[j], rv[j])

                @pl.when(cc + 2 < nch)
                def _():
                    idx_load(cc + 2, b)

    pltpu.sync_copy(acc.at[pl.ds(0, BINW)], out_hbm.at[pl.ds(BINW * w, BINW)])


def _segment_max(table, qsrc, qdst, qcnt, D):
    mesh = plsc.VectorSubcoreMesh(core_axis_name="c", subcore_axis_name="s")
    body = functools.partial(_seg_body, D=D)
    return pl.kernel(
        body,
        compiler_params=_SC_PARAMS,
        out_type=jax.ShapeDtypeStruct((NPAD, D), jnp.float32),
        mesh=mesh,
        scratch_types=[
            pltpu.VMEM((2, KROW), jnp.int32),
            pltpu.VMEM((2, KROW), jnp.int32),
            pltpu.VMEM((2, KROW, D), jnp.float32),
            pltpu.VMEM((BINW + 1, D), jnp.float32),
            pltpu.VMEM((16,), jnp.int32),
            pltpu.SemaphoreType.DMA,
            pltpu.SemaphoreType.DMA,
            pltpu.SemaphoreType.DMA,
            pltpu.SemaphoreType.DMA,
        ],
    )(table, qsrc, qdst, qcnt)


def _stage1_body(f_ref, wp_ref, bp_ref, ws_ref, bs_ref, hpool_ref, self1_ref):
    f = f_ref[...]
    hpool_ref[...] = jax.nn.relu(
        jnp.dot(f, wp_ref[...], preferred_element_type=jnp.float32) + bp_ref[...]
    )
    self1_ref[...] = jnp.dot(f, ws_ref[...], preferred_element_type=jnp.float32) + bs_ref[...]


def _stage2_body(self1_ref, neigh1_ref, wn1_ref, bn1_ref, wp2_ref, bp2_ref,
                 ws2_ref, bs2_ref, hpool2_ref, self2_ref):
    h = jax.nn.relu(
        self1_ref[...]
        + jnp.dot(neigh1_ref[...], wn1_ref[...], preferred_element_type=jnp.float32)
        + bn1_ref[...]
    )
    hpool2_ref[...] = jax.nn.relu(
        jnp.dot(h, wp2_ref[...], preferred_element_type=jnp.float32) + bp2_ref[...]
    )
    self2_ref[...] = jnp.dot(h, ws2_ref[...], preferred_element_type=jnp.float32) + bs2_ref[...]


def _stage3_body(self2_ref, neigh2_ref, wn2_ref, bn2_ref, out_ref):
    out_ref[...] = (
        self2_ref[...]
        + jnp.dot(neigh2_ref[...], wn2_ref[...], preferred_element_type=jnp.float32)
        + bn2_ref[...]
    )


def kernel(features, edge_index, W_pool1, b_pool1, W_self1, b_self1, W_neigh1, b_neigh1,
           W_pool2, b_pool2, W_self2, b_self2, W_neigh2, b_neigh2):
    src = edge_index[0]
    dst = edge_index[1]

    qsrc, qdst, qcnt = _bin_edges(src, dst)

    hpool1, self1 = pl.pallas_call(
        _stage1_body,
        out_shape=(
            jax.ShapeDtypeStruct((N, D_IN), jnp.float32),
            jax.ShapeDtypeStruct((N, D_HID), jnp.float32),
        ),
    )(features, W_pool1, b_pool1, W_self1, b_self1)

    neigh1 = _segment_max(hpool1, qsrc, qdst, qcnt, D_IN)[:N]

    hpool2, self2 = pl.pallas_call(
        _stage2_body,
        out_shape=(
            jax.ShapeDtypeStruct((N, D_HID), jnp.float32),
            jax.ShapeDtypeStruct((N, D_OUT), jnp.float32),
        ),
    )(self1, neigh1, W_neigh1, b_neigh1, W_pool2, b_pool2, W_self2, b_self2)

    neigh2 = _segment_max(hpool2, qsrc, qdst, qcnt, D_HID)[:N]

    out = pl.pallas_call(
        _stage3_body,
        out_shape=jax.ShapeDtypeStruct((N, D_OUT), jnp.float32),
    )(self2, neigh2, W_neigh2, b_neigh2)
    return out


# speedup vs baseline: 5.2827x; 1.1297x over previous
import functools

import jax
import jax.numpy as jnp
from jax import lax
from jax.experimental import pallas as pl
from jax.experimental.pallas import tpu as pltpu
from jax.experimental.pallas import tpu_sc as plsc

N = 10000
E = 320000
D_IN = 128
D_HID = 16
D_OUT = 40

NC = 2
NS = 16
NW = NC * NS
BINW = 320
NPAD = NW * BINW
MAGIC = 3277
MSHIFT = 20
SLOT = 16384
CHUNK = 20000
KROW = 128


def _worker_id():
    return lax.axis_index("s") * NC + lax.axis_index("c")


_GDN = lax.GatherDimensionNumbers(
    offset_dims=(), collapsed_slice_dims=(0,), start_index_map=(0,))


def _take16(x, idx):
    return lax.gather(x, idx[:, None], _GDN, slice_sizes=(1,),
                      mode=lax.GatherScatterMode.PROMISE_IN_BOUNDS)


BCAP = 448
SC_E = E // NW


def _bin_body(src_hbm, dst_hbm, bsrc_hbm, bdst_hbm, bcnt_hbm,
              sbuf, dbuf, bsrc, bdst, cnt32, cntv):
    w = _worker_id()
    iota = lax.iota(jnp.int32, 16)
    zv = jnp.zeros((16,), jnp.int32)

    for half in (0, 1):
        @pl.loop(0, 16 * BCAP // 16)
        def _jinit(i):
            b = (half * 16 * BCAP + i * 16) // BCAP
            bsrc[pl.ds(half * 16 * BCAP + i * 16, 16)] = zv
            bdst[pl.ds(half * 16 * BCAP + i * 16, 16)] = zv + (320 * b + 320)

    cnt32[pl.ds(0, 16)] = zv
    cnt32[pl.ds(16, 16)] = zv

    pltpu.sync_copy(src_hbm.at[pl.ds(w * SC_E, SC_E)], sbuf)
    pltpu.sync_copy(dst_hbm.at[pl.ds(w * SC_E, SC_E)], dbuf)

    @pl.loop(0, SC_E // 16)
    def _scan(i):
        d = dbuf[pl.ds(i * 16, 16)]
        s = sbuf[pl.ds(i * 16, 16)]
        b = (d * MAGIC) >> MSHIFT
        kb, perm = plsc.sort_key_val(b, iota)
        sp = _take16(s, perm)
        dp = _take16(d, perm)
        kprev = _take16(kb, jnp.maximum(iota - 1, 0))
        m0 = (kb != kprev) | (iota == 0)
        first_idx = plsc.cummax(jnp.where(m0, iota, 0))
        rank = iota - first_idx
        base = plsc.load_gather(cnt32, [kb])
        ofs = base + rank
        ok = ofs < BCAP
        pos = kb * BCAP + ofs
        plsc.store_scatter(bsrc, [pos], sp, mask=ok)
        plsc.store_scatter(bdst, [pos], dp, mask=ok)
        m0i = m0.astype(jnp.int32)
        mnext = _take16(m0i, jnp.minimum(iota + 1, 15))
        is_last = (iota == 15) | (mnext == 1)
        plsc.store_scatter(cnt32, [kb], jnp.minimum(ofs + 1, BCAP), mask=is_last)

    pltpu.sync_copy(bsrc, bsrc_hbm.at[pl.ds(w * 32 * BCAP, 32 * BCAP)])
    pltpu.sync_copy(bdst, bdst_hbm.at[pl.ds(w * 32 * BCAP, 32 * BCAP)])
    cntv[pl.ds(0, 16)] = cnt32[pl.ds(0, 16)]
    cntv[pl.ds(16, 16)] = cnt32[pl.ds(16, 16)]
    pltpu.sync_copy(cntv, bcnt_hbm.at[pl.ds(w * 32, 32)])


_SC_PARAMS = pltpu.CompilerParams(needs_layout_passes=False, use_tc_tiling_on_sc=False)


def _bin_edges(src, dst):
    mesh = plsc.VectorSubcoreMesh(core_axis_name="c", subcore_axis_name="s")
    return pl.kernel(
        _bin_body,
        compiler_params=_SC_PARAMS,
        out_type=(
            jax.ShapeDtypeStruct((NW * 32 * BCAP,), jnp.int32),
            jax.ShapeDtypeStruct((NW * 32 * BCAP,), jnp.int32),
            jax.ShapeDtypeStruct((NW * 32,), jnp.int32),
        ),
        mesh=mesh,
        scratch_types=[
            pltpu.VMEM((SC_E,), jnp.int32),
            pltpu.VMEM((SC_E,), jnp.int32),
            pltpu.VMEM((32 * BCAP,), jnp.int32),
            pltpu.VMEM((32 * BCAP,), jnp.int32),
            pltpu.VMEM((32,), jnp.int32),
            pltpu.VMEM((32,), jnp.int32),
        ],
    )(src, dst)


def _compact_body(bsrc_hbm, bdst_hbm, bcnt_hbm, qsrc_hbm, qdst_hbm, qcnt_hbm,
                  stage_s, stage_d, cmat, qsrc, qdst, cntv, sems, semd):
    w = _worker_id()
    iota = lax.iota(jnp.int32, 16)
    zv = jnp.zeros((16,), jnp.int32)
    jdv = zv + (320 * w + 320)

    @pl.loop(0, SLOT // 16)
    def _jinit(i):
        qsrc[pl.ds(i * 16, 16)] = zv
        qdst[pl.ds(i * 16, 16)] = jdv

    pltpu.sync_copy(bcnt_hbm, cmat)
    lane = w & 15
    wh = w & 16

    off = 0
    for half in (0, 1):
        for sl in range(16):
            s = half * 16 + sl
            pltpu.async_copy(
                bsrc_hbm.at[pl.ds((s * 32 + w) * BCAP, BCAP)], stage_s.at[sl], sems)
            pltpu.async_copy(
                bdst_hbm.at[pl.ds((s * 32 + w) * BCAP, BCAP)], stage_d.at[sl], semd)
        for sl in range(16):
            pltpu.make_async_copy(
                bsrc_hbm.at[pl.ds(0, BCAP)], stage_s.at[sl], sems).wait()
            pltpu.make_async_copy(
                bdst_hbm.at[pl.ds(0, BCAP)], stage_d.at[sl], semd).wait()
        for sl in range(16):
            s = half * 16 + sl
            cvec = cmat[pl.ds(s * 32 + wh, 16)]
            cb = jnp.sum(jnp.where(iota == lane, cvec, 0), axis=0)
            nv = (cb + 15) >> 4
            off_now = off

            @pl.loop(0, nv)
            def _copy(v):
                qsrc[pl.ds(off_now + v * 16, 16)] = stage_s[sl, pl.ds(v * 16, 16)]
                qdst[pl.ds(off_now + v * 16, 16)] = stage_d[sl, pl.ds(v * 16, 16)]

            off = off_now + nv * 16

    cntv[pl.ds(0, 16)] = zv + off
    pltpu.sync_copy(qsrc, qsrc_hbm.at[pl.ds(w * SLOT, SLOT)])
    pltpu.sync_copy(qdst, qdst_hbm.at[pl.ds(w * SLOT, SLOT)])
    pltpu.sync_copy(cntv, qcnt_hbm.at[pl.ds(w * 16, 16)])


def _compact(bsrc, bdst, bcnt):
    mesh = plsc.VectorSubcoreMesh(core_axis_name="c", subcore_axis_name="s")
    return pl.kernel(
        _compact_body,
        compiler_params=_SC_PARAMS,
        out_type=(
            jax.ShapeDtypeStruct((NW * SLOT,), jnp.int32),
            jax.ShapeDtypeStruct((NW * SLOT,), jnp.int32),
            jax.ShapeDtypeStruct((NW * 16,), jnp.int32),
        ),
        mesh=mesh,
        scratch_types=[
            pltpu.VMEM((16, BCAP), jnp.int32),
            pltpu.VMEM((16, BCAP), jnp.int32),
            pltpu.VMEM((NW * 32,), jnp.int32),
            pltpu.VMEM((SLOT,), jnp.int32),
            pltpu.VMEM((SLOT,), jnp.int32),
            pltpu.VMEM((16,), jnp.int32),
            pltpu.SemaphoreType.DMA,
            pltpu.SemaphoreType.DMA,
        ],
    )(bsrc, bdst, bcnt)


def _seg_body(table_hbm, qsrc_hbm, qdst_hbm, qcnt_hbm, out_hbm,
              qs, qd, rows, acc, cntv, rsem0, rsem1, isem0, isem1, *, D):
    DV = D // 16
    w = _worker_id()
    rsems = (rsem0, rsem1)
    isems = (isem0, isem1)

    @pl.loop(0, BINW + 1)
    def _zinit(r):
        for j in range(DV):
            acc[r, pl.ds(j * 16, 16)] = jnp.zeros((16,), jnp.float32)

    pltpu.sync_copy(qcnt_hbm.at[pl.ds(w * 16, 16)], cntv)
    cnt = cntv[pl.ds(0, 16)][0]
    nch = (cnt + KROW - 1) >> 7

    def idx_load(cc, b):
        pltpu.async_copy(qsrc_hbm.at[pl.ds(w * SLOT + cc * KROW, KROW)], qs.at[b], isems[b])
        pltpu.async_copy(qdst_hbm.at[pl.ds(w * SLOT + cc * KROW, KROW)], qd.at[b], isems[b])

    def idx_wait(b):
        pltpu.make_async_copy(qsrc_hbm.at[pl.ds(0, KROW)], qs.at[b], isems[b]).wait()
        pltpu.make_async_copy(qdst_hbm.at[pl.ds(0, KROW)], qd.at[b], isems[b]).wait()

    def gather_fire(b):
        pltpu.async_copy(table_hbm.at[qs.at[b]], rows.at[b], rsems[b])

    def gather_wait(b):
        pltpu.make_async_copy(table_hbm.at[qs.at[b]], rows.at[b], rsems[b]).wait()

    @pl.when(nch > 0)
    def _():
        idx_load(0, 0)

    @pl.when(nch > 1)
    def _():
        idx_load(1, 1)

    @pl.when(nch > 0)
    def _():
        idx_wait(0)
        gather_fire(0)

    nch2 = ((nch + 1) >> 1) << 1

    @pl.loop(0, nch2, step=2)
    def _outer(c2):
        for b in (0, 1):
            cc = c2 + b

            @pl.when(cc < nch)
            def _():
                gather_wait(b)

                @pl.when(cc + 1 < nch)
                def _():
                    idx_wait(1 - b)
                    gather_fire(1 - b)

                @pl.loop(0, KROW // 16)
                def _vec(v):
                    ldv = qd[b, pl.ds(v * 16, 16)] - BINW * w
                    ev = cc * KROW + v * 16 + lax.iota(jnp.int32, 16)
                    ldv = jnp.where(ev < cnt, ldv, BINW)
                    for l in range(16):
                        ld = ldv[l]
                        e = v * 16 + l
                        cur = [acc[ld, pl.ds(j * 16, 16)] for j in range(DV)]
                        rv = [rows[b, e, pl.ds(j * 16, 16)] for j in range(DV)]
                        for j in range(DV):
                            acc[ld, pl.ds(j * 16, 16)] = jnp.maximum(cur[j], rv[j])

                @pl.when(cc + 2 < nch)
                def _():
                    idx_load(cc + 2, b)

    pltpu.sync_copy(acc.at[pl.ds(0, BINW)], out_hbm.at[pl.ds(BINW * w, BINW)])


def _segment_max(table, qsrc, qdst, qcnt, D):
    mesh = plsc.VectorSubcoreMesh(core_axis_name="c", subcore_axis_name="s")
    body = functools.partial(_seg_body, D=D)
    return pl.kernel(
        body,
        compiler_params=_SC_PARAMS,
        out_type=jax.ShapeDtypeStruct((NPAD, D), jnp.float32),
        mesh=mesh,
        scratch_types=[
            pltpu.VMEM((2, KROW), jnp.int32),
            pltpu.VMEM((2, KROW), jnp.int32),
            pltpu.VMEM((2, KROW, D), jnp.float32),
            pltpu.VMEM((BINW + 1, D), jnp.float32),
            pltpu.VMEM((16,), jnp.int32),
            pltpu.SemaphoreType.DMA,
            pltpu.SemaphoreType.DMA,
            pltpu.SemaphoreType.DMA,
            pltpu.SemaphoreType.DMA,
        ],
    )(table, qsrc, qdst, qcnt)


def _stage1_body(f_ref, wp_ref, bp_ref, ws_ref, bs_ref, hpool_ref, self1_ref):
    f = f_ref[...]
    hpool_ref[...] = jax.nn.relu(
        jnp.dot(f, wp_ref[...], preferred_element_type=jnp.float32) + bp_ref[...]
    )
    self1_ref[...] = jnp.dot(f, ws_ref[...], preferred_element_type=jnp.float32) + bs_ref[...]


def _stage2_body(self1_ref, neigh1_ref, wn1_ref, bn1_ref, wp2_ref, bp2_ref,
                 ws2_ref, bs2_ref, hpool2_ref, self2_ref):
    h = jax.nn.relu(
        self1_ref[...]
        + jnp.dot(neigh1_ref[...], wn1_ref[...], preferred_element_type=jnp.float32)
        + bn1_ref[...]
    )
    hpool2_ref[...] = jax.nn.relu(
        jnp.dot(h, wp2_ref[...], preferred_element_type=jnp.float32) + bp2_ref[...]
    )
    self2_ref[...] = jnp.dot(h, ws2_ref[...], preferred_element_type=jnp.float32) + bs2_ref[...]


def _stage3_body(self2_ref, neigh2_ref, wn2_ref, bn2_ref, out_ref):
    out_ref[...] = (
        self2_ref[...]
        + jnp.dot(neigh2_ref[...], wn2_ref[...], preferred_element_type=jnp.float32)
        + bn2_ref[...]
    )


def kernel(features, edge_index, W_pool1, b_pool1, W_self1, b_self1, W_neigh1, b_neigh1,
           W_pool2, b_pool2, W_self2, b_self2, W_neigh2, b_neigh2):
    src = edge_index[0]
    dst = edge_index[1]

    bsrc, bdst, bcnt = _bin_edges(src, dst)
    qsrc, qdst, qcnt = _compact(bsrc, bdst, bcnt)

    hpool1, self1 = pl.pallas_call(
        _stage1_body,
        out_shape=(
            jax.ShapeDtypeStruct((N, D_IN), jnp.float32),
            jax.ShapeDtypeStruct((N, D_HID), jnp.float32),
        ),
    )(features, W_pool1, b_pool1, W_self1, b_self1)

    neigh1 = _segment_max(hpool1, qsrc, qdst, qcnt, D_IN)[:N]

    hpool2, self2 = pl.pallas_call(
        _stage2_body,
        out_shape=(
            jax.ShapeDtypeStruct((N, D_HID), jnp.float32),
            jax.ShapeDtypeStruct((N, D_OUT), jnp.float32),
        ),
    )(self1, neigh1, W_neigh1, b_neigh1, W_pool2, b_pool2, W_self2, b_self2)

    neigh2 = _segment_max(hpool2, qsrc, qdst, qcnt, D_HID)[:N]

    out = pl.pallas_call(
        _stage3_body,
        out_shape=jax.ShapeDtypeStruct((N, D_OUT), jnp.float32),
    )(self2, neigh2, W_neigh2, b_neigh2)
    return out
